# baseline (speedup 1.0000x reference)
import functools

import jax
import jax.numpy as jnp
from jax import lax
from jax.experimental import pallas as pl
from jax.experimental.pallas import tpu as pltpu
from jax.experimental.pallas import tpu_sc as plsc


B, N, E, ATOM_DIM, BOND_DIM = 64, 128, 256, 64, 16
C = 16
BF = jnp.bfloat16

NW = 32
ROWS_PER_W = B * E // NW
CHUNK = 128
NCHUNK = ROWS_PER_W // CHUNK
BATCH_PER_W = B // NW
ACC_ROWS = BATCH_PER_W * N

_MESH = plsc.VectorSubcoreMesh(core_axis_name="c", subcore_axis_name="s")


@functools.partial(
    pl.kernel, mesh=_MESH,
    compiler_params=pltpu.CompilerParams(use_tc_tiling_on_sc=False),
    out_type=jax.ShapeDtypeStruct((B * E, ATOM_DIM), jnp.float32),
    scratch_types=[
        pltpu.VMEM((NCHUNK, CHUNK), jnp.int32),
        pltpu.VMEM((ROWS_PER_W, ATOM_DIM), jnp.float32),
        pltpu.SemaphoreType.DMA,
    ],
)
def _sc_gather(table_hbm, gidx_hbm, out_hbm, idx_v, rows_v, sem):
    wid = lax.axis_index("s") * 2 + lax.axis_index("c")
    pltpu.sync_copy(gidx_hbm.at[wid], idx_v)
    for j in range(NCHUNK):
        pltpu.async_copy(table_hbm.at[idx_v.at[j]],
                         rows_v.at[pl.ds(j * CHUNK, CHUNK)], sem).wait()
    pltpu.sync_copy(rows_v, out_hbm.at[pl.ds(wid * ROWS_PER_W, ROWS_PER_W)])


@functools.partial(
    pl.kernel, mesh=_MESH,
    compiler_params=pltpu.CompilerParams(use_tc_tiling_on_sc=False),
    out_type=jax.ShapeDtypeStruct((B * N, ATOM_DIM), jnp.float32),
    scratch_types=[
        pltpu.VMEM((NCHUNK, CHUNK), jnp.int32),
        pltpu.VMEM((ROWS_PER_W, ATOM_DIM), jnp.float32),
        pltpu.VMEM((ACC_ROWS, ATOM_DIM), jnp.float32),
        pltpu.VMEM_SHARED((16 * ACC_ROWS, ATOM_DIM), jnp.float32),
        pltpu.SemaphoreType.DMA,
    ],
)
def _sc_scatter(msg_hbm, sidx_hbm, out_hbm, idx_v, msg_v, zbuf_v, acc_sh,
                sem):
    cid = lax.axis_index("c")
    sid = lax.axis_index("s")
    wid = sid * 2 + cid
    pltpu.sync_copy(sidx_hbm.at[wid], idx_v)
    pltpu.sync_copy(msg_hbm.at[pl.ds(wid * ROWS_PER_W, ROWS_PER_W)], msg_v)

    zero = jnp.zeros((16,), jnp.float32)

    def _zero_row(i, carry):
        for cb in range(ATOM_DIM // 16):
            zbuf_v[i, pl.ds(cb * 16, 16)] = zero
        return carry

    lax.fori_loop(0, ACC_ROWS, _zero_row, 0)
    pltpu.sync_copy(zbuf_v, acc_sh.at[pl.ds(sid * ACC_ROWS, ACC_ROWS)])

    for j in range(NCHUNK):
        pltpu.sync_copy(msg_v.at[pl.ds(j * CHUNK, CHUNK)],
                        acc_sh.at[idx_v.at[j]], add=True)
    pltpu.sync_copy(acc_sh.at[pl.ds(sid * ACC_ROWS, ACC_ROWS)],
                    out_hbm.at[pl.ds(wid * ACC_ROWS, ACC_ROWS)])


def _tc_kernel(src_ref, bond_ref, w_ref, r_ref, out_ref):
    src = src_ref[...].astype(BF)
    bond = bond_ref[...].astype(BF)
    bond_exp = jax.lax.dot(bond, r_ref[...],
                           preferred_element_type=jnp.float32).astype(BF)
    g = jnp.tile(src, (1, BOND_DIM)) * bond_exp
    out_ref[...] = jax.lax.dot(g, w_ref[...],
                               preferred_element_type=jnp.float32)


@jax.jit
def kernel(atom_state, bond_state, connectivity, bond_transform):
    barange = jnp.arange(B, dtype=jnp.int32)[:, None]
    gidx = (connectivity[:, :, 0] + barange * N).reshape(NW, NCHUNK, CHUNK)
    sid_of_b = barange // (2 * BATCH_PER_W)
    sidx = (sid_of_b * ACC_ROWS + (barange % BATCH_PER_W) * N
            + connectivity[:, :, 1]).reshape(NW, NCHUNK, CHUNK)
    w = bond_transform.reshape(BOND_DIM, ATOM_DIM, ATOM_DIM)
    w = w.transpose(0, 2, 1).reshape(BOND_DIM * ATOM_DIM, ATOM_DIM)
    r = jnp.repeat(jnp.eye(BOND_DIM, dtype=BF), ATOM_DIM, axis=1)

    src_atoms = _sc_gather(atom_state.reshape(B * N, ATOM_DIM), gidx)

    rows = C * E
    msg = pl.pallas_call(
        _tc_kernel,
        grid=(B * E // rows,),
        in_specs=[
            pl.BlockSpec((rows, ATOM_DIM), lambda b: (b, 0)),
            pl.BlockSpec((rows, BOND_DIM), lambda b: (b, 0)),
            pl.BlockSpec((BOND_DIM * ATOM_DIM, ATOM_DIM), lambda b: (0, 0)),
            pl.BlockSpec((BOND_DIM, BOND_DIM * ATOM_DIM), lambda b: (0, 0)),
        ],
        out_specs=pl.BlockSpec((rows, ATOM_DIM), lambda b: (b, 0)),
        out_shape=jax.ShapeDtypeStruct((B * E, ATOM_DIM), jnp.float32),
    )(src_atoms, bond_state.reshape(B * E, BOND_DIM), w.astype(BF), r)

    out = _sc_scatter(msg, sidx)
    return out.reshape(B, N, ATOM_DIM)


# speedup vs baseline: 2.1638x; 2.1638x over previous
import jax
import jax.numpy as jnp
from jax.experimental import pallas as pl


B, N, E, ATOM_DIM, BOND_DIM = 64, 128, 256, 64, 16
C = 16
BF = jnp.bfloat16


def _bmm_kernel(atom_ref, bond_ref, src_ref, tgt_ref, w_ref, r_ref,
                out_ref):
    w = w_ref[...]

    iota_n = jax.lax.broadcasted_iota(jnp.int32, (E, N), 1)
    gathered = []
    for c in range(C):
        oh_src = (iota_n == src_ref[c, 0][:, None]).astype(BF)
        gathered.append(jax.lax.dot(oh_src, atom_ref[c],
                                    preferred_element_type=jnp.float32))
    src_atoms = jnp.concatenate(gathered, axis=0).astype(BF)

    bond = bond_ref[...].reshape(C * E, BOND_DIM)
    bond_exp = jax.lax.dot(bond, r_ref[...],
                           preferred_element_type=jnp.float32).astype(BF)
    g = jnp.tile(src_atoms, (1, BOND_DIM)) * bond_exp
    msg = jax.lax.dot(g, w, preferred_element_type=jnp.float32)
    msg = msg.astype(BF)

    iota_t = jax.lax.broadcasted_iota(jnp.int32, (N, E), 0)
    for c in range(C):
        oh_tgt = (iota_t == tgt_ref[c, 0][None, :]).astype(BF)
        out_ref[c] = jax.lax.dot(oh_tgt, msg[c * E:(c + 1) * E],
                                 preferred_element_type=jnp.float32)


@jax.jit
def kernel(atom_state, bond_state, connectivity, bond_transform):
    w = bond_transform.reshape(BOND_DIM, ATOM_DIM, ATOM_DIM)
    w = w.transpose(0, 2, 1).reshape(BOND_DIM * ATOM_DIM, ATOM_DIM)
    src_idx = connectivity[:, :, 0].reshape(B, 1, E)
    tgt_idx = connectivity[:, :, 1].reshape(B, 1, E)
    r = jnp.repeat(jnp.eye(BOND_DIM, dtype=BF), ATOM_DIM, axis=1)

    return pl.pallas_call(
        _bmm_kernel,
        grid=(B // C,),
        in_specs=[
            pl.BlockSpec((C, N, ATOM_DIM), lambda b: (b, 0, 0)),
            pl.BlockSpec((C, E, BOND_DIM), lambda b: (b, 0, 0)),
            pl.BlockSpec((C, 1, E), lambda b: (b, 0, 0)),
            pl.BlockSpec((C, 1, E), lambda b: (b, 0, 0)),
            pl.BlockSpec((BOND_DIM * ATOM_DIM, ATOM_DIM), lambda b: (0, 0)),
            pl.BlockSpec((BOND_DIM, BOND_DIM * ATOM_DIM), lambda b: (0, 0)),
        ],
        out_specs=pl.BlockSpec((C, N, ATOM_DIM), lambda b: (b, 0, 0)),
        out_shape=jax.ShapeDtypeStruct((B, N, ATOM_DIM), jnp.float32),
    )(atom_state.astype(BF), bond_state.astype(BF), src_idx, tgt_idx,
      w.astype(BF), r)
